# Initial kernel scaffold; baseline (speedup 1.0000x reference)
import jax
import jax.numpy as jnp
from jax.experimental import pallas as pl


def kernel(node_features, W_in, b_in, W0, b0, g0, be0, W1, b1, g1, be1, W2, b2, g2, be2, W_out, b_out, edge_index):
    raise NotImplementedError("write your pallas kernel here")


# speedup vs baseline: 6.3315x; 6.3315x over previous
import functools

import jax
import jax.numpy as jnp
from jax import lax
from jax.experimental import pallas as pl
from jax.experimental.pallas import tpu as pltpu
from jax.experimental.pallas import tpu_sc as plsc

N = 10000
E = 320000
D = 128

NC = 2
NS = 16
NW = NC * NS

EPT = E // NW
CH = 80
NCH = EPT // CH
CU = 16
NCU = N // CU
CPT = NCU // NS

_BN = 1000


def _agg_kernel_entry(table, srcs, dsts, zeros, out, src_v, dst_v, rows_v,
                      zbuf, acc, sem):
    cid = lax.axis_index("c")
    sid = lax.axis_index("s")
    tid = cid * NS + sid

    base_c = sid * CPT
    n_c = CPT + jnp.where(sid == NS - 1, NCU - NS * CPT, 0)

    pltpu.sync_copy(zeros, zbuf)

    def zbody(j, carry):
        r = pl.multiple_of((base_c + j) * CU, CU)
        pltpu.sync_copy(zbuf, acc.at[pl.ds(r, CU)])
        return carry

    lax.fori_loop(0, n_c, zbody, 0)

    pltpu.sync_copy(srcs.at[tid], src_v)
    pltpu.sync_copy(dsts.at[tid], dst_v)
    plsc.subcore_barrier()

    def body(j, carry):
        pltpu.async_copy(table.at[src_v.at[j]], rows_v, sem).wait()
        pltpu.sync_copy(rows_v, acc.at[dst_v.at[j]], add=True)
        return carry

    lax.fori_loop(0, NCH, body, 0)
    plsc.subcore_barrier()

    def obody(j, carry):
        r = pl.multiple_of((base_c + j) * CU, CU)
        pltpu.sync_copy(acc.at[pl.ds(r, CU)], zbuf)
        pltpu.sync_copy(zbuf, out.at[cid, pl.ds(r, CU)])
        return carry

    lax.fori_loop(0, n_c, obody, 0)


@jax.jit
def _aggregate(table, srcs, dsts, zeros):
    mesh = plsc.VectorSubcoreMesh(core_axis_name="c", subcore_axis_name="s")
    k = functools.partial(
        pl.kernel,
        mesh=mesh,
        out_type=jax.ShapeDtypeStruct((NC, N, D), jnp.float32),
        scratch_types=[
            pltpu.VMEM((NCH, CH), jnp.int32),
            pltpu.VMEM((NCH, CH), jnp.int32),
            pltpu.VMEM((CH, D), jnp.float32),
            pltpu.VMEM((CU, D), jnp.float32),
            pltpu.VMEM_SHARED((N, D), jnp.float32),
            pltpu.SemaphoreType.DMA,
        ],
    )(_agg_kernel_entry)
    return k(table, srcs, dsts, zeros)


def _pre_body(x_ref, wint_ref, bin_ref, w0_ref, h_ref, hw_ref):
    h = jnp.dot(x_ref[...], wint_ref[...],
                preferred_element_type=jnp.float32) + bin_ref[...]
    h_ref[...] = h
    hw_ref[...] = jnp.dot(h, w0_ref[...], preferred_element_type=jnp.float32)


def _ln(h, g, be):
    mu = jnp.mean(h, axis=-1, keepdims=True)
    var = jnp.mean((h - mu) ** 2, axis=-1, keepdims=True)
    return (h - mu) * lax.rsqrt(var + 1e-5) * g + be


def _mid_body(p_ref, b_ref, res_ref, g_ref, be_ref, wn_ref, h_ref, hw_ref):
    s = p_ref[0] + p_ref[1] + b_ref[...]
    h = jnp.maximum(s, 0.0) + res_ref[...]
    hn = _ln(h, g_ref[...], be_ref[...])
    h_ref[...] = hn
    hw_ref[...] = jnp.dot(hn, wn_ref[...], preferred_element_type=jnp.float32)


def _fin_body(p_ref, b_ref, res_ref, g_ref, be_ref, wot_ref, bo_ref, o_ref):
    s = p_ref[0] + p_ref[1] + b_ref[...]
    h = jnp.maximum(s, 0.0) + res_ref[...]
    hn = _ln(h, g_ref[...], be_ref[...])
    o_ref[...] = jnp.dot(hn, wot_ref[...],
                         preferred_element_type=jnp.float32) + bo_ref[...]


_row_spec = pl.BlockSpec((_BN, D), lambda i: (i, 0))
_mat_spec = pl.BlockSpec((D, D), lambda i: (0, 0))
_vec_spec = pl.BlockSpec((1, D), lambda i: (0, 0))
_par_spec = pl.BlockSpec((NC, _BN, D), lambda i: (0, i, 0))
_out2 = [jax.ShapeDtypeStruct((N, D), jnp.float32)] * 2
_out1 = jax.ShapeDtypeStruct((N, D), jnp.float32)


@jax.jit
def _pre(x, wint, bin_, w0):
    return pl.pallas_call(
        _pre_body,
        grid=(N // _BN,),
        in_specs=[_row_spec, _mat_spec, _vec_spec, _mat_spec],
        out_specs=[_row_spec, _row_spec],
        out_shape=_out2,
    )(x, wint, bin_, w0)


@jax.jit
def _mid(p, b, res, g, be, wn):
    return pl.pallas_call(
        _mid_body,
        grid=(N // _BN,),
        in_specs=[_par_spec, _vec_spec, _row_spec, _vec_spec, _vec_spec,
                  _mat_spec],
        out_specs=[_row_spec, _row_spec],
        out_shape=_out2,
    )(p, b, res, g, be, wn)


@jax.jit
def _fin(p, b, res, g, be, wot, bo):
    return pl.pallas_call(
        _fin_body,
        grid=(N // _BN,),
        in_specs=[_par_spec, _vec_spec, _row_spec, _vec_spec, _vec_spec,
                  _mat_spec, _vec_spec],
        out_specs=_row_spec,
        out_shape=_out1,
    )(p, b, res, g, be, wot, bo)


def kernel(node_features, W_in, b_in, W0, b0, g0, be0, W1, b1, g1, be1,
           W2, b2, g2, be2, W_out, b_out, edge_index):
    srcs = edge_index[0].reshape(NW, NCH, CH)
    dsts = edge_index[1].reshape(NW, NCH, CH)
    zeros = jnp.zeros((CU, D), jnp.float32)

    r2 = lambda v: v.reshape(1, D)

    h, hw = _pre(node_features, W_in.T, r2(b_in), W0)

    p = _aggregate(hw, srcs, dsts, zeros)
    h, hw = _mid(p, r2(b0), h, r2(g0), r2(be0), W1)

    p = _aggregate(hw, srcs, dsts, zeros)
    h, hw = _mid(p, r2(b1), h, r2(g1), r2(be1), W2)

    p = _aggregate(hw, srcs, dsts, zeros)
    return _fin(p, r2(b2), h, r2(g2), r2(be2), W_out.T, r2(b_out))


# speedup vs baseline: 9.6989x; 1.5318x over previous
import functools

import jax
import jax.numpy as jnp
from jax import lax
from jax.experimental import pallas as pl
from jax.experimental.pallas import tpu as pltpu
from jax.experimental.pallas import tpu_sc as plsc

N = 10000
E = 320000
D = 128

NC = 2
NS = 16
NW = NC * NS

EPT = E // NW
CH = 80
NCH = EPT // CH
CU = 16
NCU = N // CU
CPT = NCU // NS

_BN = 1000


def _agg_kernel_entry(table, srcs, dsts, zeros, out, sbuf0, sbuf1, dst_v,
                      rows0, rows1, acc, sem0, sem1, sems0, sems1):
    cid = lax.axis_index("c")
    sid = lax.axis_index("s")
    tid = cid * NS + sid

    base_c = sid * CPT
    n_c = CPT + jnp.where(sid == NS - 1, NCU - NS * CPT, 0)
    ebase = tid * EPT

    def soff(j):
        return pl.multiple_of(ebase + j * CH, 8)

    zb = rows0.at[pl.ds(0, CU)]
    pltpu.sync_copy(zeros, zb)

    def zbody(j, carry):
        r = pl.multiple_of((base_c + j) * CU, CU)
        pltpu.sync_copy(zb, acc.at[pl.ds(r, CU)])
        return carry

    lax.fori_loop(0, n_c, zbody, 0)

    pltpu.sync_copy(dsts.at[tid], dst_v)
    plsc.subcore_barrier()

    pltpu.sync_copy(srcs.at[pl.ds(soff(0), CH)], sbuf0)
    pltpu.sync_copy(srcs.at[pl.ds(soff(1), CH)], sbuf1)
    g0 = pltpu.async_copy(table.at[sbuf0], rows0, sem0)
    g1 = pltpu.async_copy(table.at[sbuf1], rows1, sem1)

    def body(t, carry):
        j0 = 2 * t
        g0.wait()
        s0 = pltpu.async_copy(srcs.at[pl.ds(soff(j0 + 2), CH)], sbuf0, sems0)
        pltpu.sync_copy(rows0, acc.at[dst_v.at[j0]], add=True)
        s0.wait()
        pltpu.async_copy(table.at[sbuf0], rows0, sem0)

        g1.wait()

        @pl.when(t < NCH // 2 - 1)
        def _():
            s1 = pltpu.async_copy(srcs.at[pl.ds(soff(j0 + 3), CH)], sbuf1,
                                  sems1)
            pltpu.sync_copy(rows1, acc.at[dst_v.at[j0 + 1]], add=True)
            s1.wait()
            pltpu.async_copy(table.at[sbuf1], rows1, sem1)

        @pl.when(t >= NCH // 2 - 1)
        def _():
            pltpu.sync_copy(rows1, acc.at[dst_v.at[j0 + 1]], add=True)

        return carry

    lax.fori_loop(0, NCH // 2, body, 0)
    g0.wait()
    pltpu.sync_copy(rows0, acc.at[dst_v.at[NCH - 1]], add=True)
    plsc.subcore_barrier()

    zb2 = rows0.at[pl.ds(0, CU)]

    def obody(j, carry):
        r = pl.multiple_of((base_c + j) * CU, CU)
        pltpu.sync_copy(acc.at[pl.ds(r, CU)], zb2)
        pltpu.sync_copy(zb2, out.at[cid, pl.ds(r, CU)])
        return carry

    lax.fori_loop(0, n_c, obody, 0)


@jax.jit
def _aggregate(table, srcs, dsts, zeros):
    mesh = plsc.VectorSubcoreMesh(core_axis_name="c", subcore_axis_name="s")
    k = functools.partial(
        pl.kernel,
        mesh=mesh,
        out_type=jax.ShapeDtypeStruct((NC, N, D), jnp.float32),
        scratch_types=[
            pltpu.VMEM((CH,), jnp.int32),
            pltpu.VMEM((CH,), jnp.int32),
            pltpu.VMEM((NCH, CH), jnp.int32),
            pltpu.VMEM((CH, D), jnp.float32),
            pltpu.VMEM((CH, D), jnp.float32),
            pltpu.VMEM_SHARED((N, D), jnp.float32),
            pltpu.SemaphoreType.DMA,
            pltpu.SemaphoreType.DMA,
            pltpu.SemaphoreType.DMA,
            pltpu.SemaphoreType.DMA,
        ],
    )(_agg_kernel_entry)
    return k(table, srcs, dsts, zeros)


def _pre_body(x_ref, wint_ref, bin_ref, w0_ref, h_ref, hw_ref):
    h = jnp.dot(x_ref[...], wint_ref[...],
                preferred_element_type=jnp.float32) + bin_ref[...]
    h_ref[...] = h
    hw_ref[...] = jnp.dot(h, w0_ref[...], preferred_element_type=jnp.float32)


def _ln(h, g, be):
    mu = jnp.mean(h, axis=-1, keepdims=True)
    var = jnp.mean((h - mu) ** 2, axis=-1, keepdims=True)
    return (h - mu) * lax.rsqrt(var + 1e-5) * g + be


def _mid_body(p_ref, b_ref, res_ref, g_ref, be_ref, wn_ref, h_ref, hw_ref):
    s = p_ref[0] + p_ref[1] + b_ref[...]
    h = jnp.maximum(s, 0.0) + res_ref[...]
    hn = _ln(h, g_ref[...], be_ref[...])
    h_ref[...] = hn
    hw_ref[...] = jnp.dot(hn, wn_ref[...], preferred_element_type=jnp.float32)


def _fin_body(p_ref, b_ref, res_ref, g_ref, be_ref, wot_ref, bo_ref, o_ref):
    s = p_ref[0] + p_ref[1] + b_ref[...]
    h = jnp.maximum(s, 0.0) + res_ref[...]
    hn = _ln(h, g_ref[...], be_ref[...])
    o_ref[...] = jnp.dot(hn, wot_ref[...],
                         preferred_element_type=jnp.float32) + bo_ref[...]


_row_spec = pl.BlockSpec((_BN, D), lambda i: (i, 0))
_mat_spec = pl.BlockSpec((D, D), lambda i: (0, 0))
_vec_spec = pl.BlockSpec((1, D), lambda i: (0, 0))
_par_spec = pl.BlockSpec((NC, _BN, D), lambda i: (0, i, 0))
_out2 = [jax.ShapeDtypeStruct((N, D), jnp.float32)] * 2
_out1 = jax.ShapeDtypeStruct((N, D), jnp.float32)


@jax.jit
def _pre(x, wint, bin_, w0):
    return pl.pallas_call(
        _pre_body,
        grid=(N // _BN,),
        in_specs=[_row_spec, _mat_spec, _vec_spec, _mat_spec],
        out_specs=[_row_spec, _row_spec],
        out_shape=_out2,
    )(x, wint, bin_, w0)


@jax.jit
def _mid(p, b, res, g, be, wn):
    return pl.pallas_call(
        _mid_body,
        grid=(N // _BN,),
        in_specs=[_par_spec, _vec_spec, _row_spec, _vec_spec, _vec_spec,
                  _mat_spec],
        out_specs=[_row_spec, _row_spec],
        out_shape=_out2,
    )(p, b, res, g, be, wn)


@jax.jit
def _fin(p, b, res, g, be, wot, bo):
    return pl.pallas_call(
        _fin_body,
        grid=(N // _BN,),
        in_specs=[_par_spec, _vec_spec, _row_spec, _vec_spec, _vec_spec,
                  _mat_spec, _vec_spec],
        out_specs=_row_spec,
        out_shape=_out1,
    )(p, b, res, g, be, wot, bo)


def kernel(node_features, W_in, b_in, W0, b0, g0, be0, W1, b1, g1, be1,
           W2, b2, g2, be2, W_out, b_out, edge_index):
    srcs = edge_index[0]
    dsts = edge_index[1].reshape(NW, NCH, CH)
    zeros = jnp.zeros((CU, D), jnp.float32)

    r2 = lambda v: v.reshape(1, D)

    h, hw = _pre(node_features, W_in.T, r2(b_in), W0)

    p = _aggregate(hw, srcs, dsts, zeros)
    h, hw = _mid(p, r2(b0), h, r2(g0), r2(be0), W1)

    p = _aggregate(hw, srcs, dsts, zeros)
    h, hw = _mid(p, r2(b1), h, r2(g1), r2(be1), W2)

    p = _aggregate(hw, srcs, dsts, zeros)
    return _fin(p, r2(b2), h, r2(g2), r2(be2), W_out.T, r2(b_out))
